# initial kernel scaffold (unmeasured)
import jax
import jax.numpy as jnp
from jax import lax
from jax.experimental import pallas as pl
from jax.experimental.pallas import tpu as pltpu

N_DEV = 4
M = 8192
N = 4096
CHUNK = M // N_DEV


def _body(partial_ref, out_ref, work_ref, send_ref, mine_ref, outstage_ref,
          send_sem, recv_sem, load_sem, store_sem, credit_sem):
    my = lax.axis_index("i")
    left = (my - 1) % N_DEV
    right = (my + 1) % N_DEV

    barrier = pltpu.get_barrier_semaphore()
    for nbr in (left, right):
        pl.semaphore_signal(barrier, inc=1, device_id=(nbr,),
                            device_id_type=pl.DeviceIdType.MESH)
    pl.semaphore_wait(barrier, 2)

    def rdma_to_right():
        return pltpu.make_async_remote_copy(
            src_ref=send_ref, dst_ref=work_ref,
            send_sem=send_sem, recv_sem=recv_sem,
            device_id=(right,), device_id_type=pl.DeviceIdType.MESH,
        )

    def grant_credit():
        pl.semaphore_signal(credit_sem, inc=1, device_id=(left,),
                            device_id_type=pl.DeviceIdType.MESH)

    load = pltpu.make_async_copy(
        partial_ref.at[pl.ds(my * CHUNK, CHUNK), :], send_ref, load_sem)
    load.start()
    load.wait()

    for s in range(N_DEV - 1):
        c_recv = (my - s - 1) % N_DEV
        if s >= 1:
            pl.semaphore_wait(credit_sem, 1)
        rdma = rdma_to_right()
        rdma.start()
        load = pltpu.make_async_copy(
            partial_ref.at[pl.ds(c_recv * CHUNK, CHUNK), :], mine_ref,
            load_sem)
        load.start()
        load.wait()
        rdma.wait()
        acc = work_ref[...].astype(jnp.float32) + mine_ref[...].astype(jnp.float32)
        send_ref[...] = acc.astype(jnp.bfloat16)
        grant_credit()

    own = (my + 1) % N_DEV
    y = send_ref[...].astype(jnp.float32)
    act = y * jax.nn.sigmoid(y)
    outstage_ref[...] = act
    send_ref[...] = act.astype(jnp.bfloat16)
    store = pltpu.make_async_copy(
        outstage_ref, out_ref.at[pl.ds(own * CHUNK, CHUNK), :], store_sem)
    store.start()

    for s in range(N_DEV - 1):
        c = (my - s) % N_DEV
        pl.semaphore_wait(credit_sem, 1)
        rdma = rdma_to_right()
        rdma.start()
        rdma.wait()
        if s < N_DEV - 2:
            send_ref[...] = work_ref[...]
        store.wait()
        outstage_ref[...] = work_ref[...].astype(jnp.float32)
        if s < N_DEV - 2:
            grant_credit()
        store = pltpu.make_async_copy(
            outstage_ref, out_ref.at[pl.ds(c * CHUNK, CHUNK), :], store_sem)
        store.start()
    store.wait()


def kernel(x, w_mat):
    partial = jnp.dot(
        x, w_mat, preferred_element_type=jnp.float32
    ).astype(jnp.bfloat16)

    return pl.pallas_call(
        _body,
        out_shape=jax.ShapeDtypeStruct((M, N), jnp.float32),
        in_specs=[pl.BlockSpec(memory_space=pltpu.ANY)],
        out_specs=pl.BlockSpec(memory_space=pltpu.ANY),
        scratch_shapes=[
            pltpu.VMEM((CHUNK, N), jnp.bfloat16),
            pltpu.VMEM((CHUNK, N), jnp.bfloat16),
            pltpu.VMEM((CHUNK, N), jnp.bfloat16),
            pltpu.VMEM((CHUNK, N), jnp.float32),
            pltpu.SemaphoreType.DMA,
            pltpu.SemaphoreType.DMA,
            pltpu.SemaphoreType.DMA,
            pltpu.SemaphoreType.DMA,
            pltpu.SemaphoreType.REGULAR,
        ],
        compiler_params=pltpu.CompilerParams(
            collective_id=0,
            vmem_limit_bytes=128 * 1024 * 1024,
        ),
    )(partial)


# baseline (device time: 1426450 ns/iter reference)
import jax
import jax.numpy as jnp
from jax import lax
from jax.experimental import pallas as pl
from jax.experimental.pallas import tpu as pltpu

N_DEV = 4
M = 8192
N = 4096
CHUNK = M // N_DEV
SUB = CHUNK // 4


def _body(partial_ref, out_ref, work_ref, send_ref, mine_ref, outstage_ref,
          send_sem, recv_sem, load_sem, store_sem, credit_sem):
    my = lax.axis_index("i")
    left = (my - 1) % N_DEV
    right = (my + 1) % N_DEV

    barrier = pltpu.get_barrier_semaphore()
    for nbr in (left, right):
        pl.semaphore_signal(barrier, inc=1, device_id=(nbr,),
                            device_id_type=pl.DeviceIdType.MESH)
    pl.semaphore_wait(barrier, 2)

    def rdma_to_right():
        return pltpu.make_async_remote_copy(
            src_ref=send_ref, dst_ref=work_ref,
            send_sem=send_sem, recv_sem=recv_sem,
            device_id=(right,), device_id_type=pl.DeviceIdType.MESH,
        )

    def grant_credit():
        pl.semaphore_signal(credit_sem, inc=1, device_id=(left,),
                            device_id_type=pl.DeviceIdType.MESH)

    pending_store = []

    def store_chunk(src_ref, c):
        for sub in range(CHUNK // SUB):
            if pending_store:
                pending_store.pop().wait()
            outstage_ref[...] = (
                src_ref[pl.ds(sub * SUB, SUB), :].astype(jnp.float32))
            st = pltpu.make_async_copy(
                outstage_ref,
                out_ref.at[pl.ds(c * CHUNK + sub * SUB, SUB), :],
                store_sem)
            st.start()
            pending_store.append(st)

    load = pltpu.make_async_copy(
        partial_ref.at[pl.ds(my * CHUNK, CHUNK), :], send_ref, load_sem)
    load.start()
    load.wait()

    for s in range(N_DEV - 1):
        c_recv = (my - s - 1) % N_DEV
        if s >= 1:
            pl.semaphore_wait(credit_sem, 1)
        rdma = rdma_to_right()
        rdma.start()
        load = pltpu.make_async_copy(
            partial_ref.at[pl.ds(c_recv * CHUNK, CHUNK), :], mine_ref,
            load_sem)
        load.start()
        load.wait()
        rdma.wait()
        acc = work_ref[...].astype(jnp.float32) + mine_ref[...].astype(jnp.float32)
        send_ref[...] = acc.astype(jnp.bfloat16)
        grant_credit()

    own = (my + 1) % N_DEV
    y = send_ref[...].astype(jnp.float32)
    send_ref[...] = (y * jax.nn.sigmoid(y)).astype(jnp.bfloat16)

    for s in range(N_DEV - 1):
        c = (my - s) % N_DEV
        pl.semaphore_wait(credit_sem, 1)
        rdma = rdma_to_right()
        rdma.start()
        if s == 0:
            store_chunk(send_ref, own)
        rdma.wait()
        if s < N_DEV - 2:
            send_ref[...] = work_ref[...]
        store_chunk(work_ref, c)
        if s < N_DEV - 2:
            grant_credit()
    pending_store.pop().wait()


def kernel(x, w_mat):
    partial = jnp.dot(
        x, w_mat, preferred_element_type=jnp.float32
    ).astype(jnp.bfloat16)

    return pl.pallas_call(
        _body,
        out_shape=jax.ShapeDtypeStruct((M, N), jnp.float32),
        in_specs=[pl.BlockSpec(memory_space=pl.ANY)],
        out_specs=pl.BlockSpec(memory_space=pl.ANY),
        scratch_shapes=[
            pltpu.VMEM((CHUNK, N), jnp.bfloat16),
            pltpu.VMEM((CHUNK, N), jnp.bfloat16),
            pltpu.VMEM((CHUNK, N), jnp.bfloat16),
            pltpu.VMEM((SUB, N), jnp.float32),
            pltpu.SemaphoreType.DMA,
            pltpu.SemaphoreType.DMA,
            pltpu.SemaphoreType.DMA,
            pltpu.SemaphoreType.DMA,
            pltpu.SemaphoreType.REGULAR,
        ],
        compiler_params=pltpu.CompilerParams(
            collective_id=0,
            vmem_limit_bytes=64 * 1024 * 1024,
        ),
    )(partial)


# device time: 870380 ns/iter; 1.6389x vs baseline; 1.6389x over previous
import jax
import jax.numpy as jnp
from jax import lax
from jax.experimental import pallas as pl
from jax.experimental.pallas import tpu as pltpu

N_DEV = 4
M = 8192
N = 4096
HALF = N // 2
CHUNK = M // N_DEV
SUB = 512


def _body(partial_ref, out_ref,
          work0, work1, send0, send1, mine0, mine1, outst0, outst1,
          send_sems, recv_sems, load_sems, store_sems, credit0, credit1):
    my = lax.axis_index("i")
    left = (my - 1) % N_DEV
    right = (my + 1) % N_DEV

    load_a = pltpu.make_async_copy(
        partial_ref.at[pl.ds(my * CHUNK, CHUNK), pl.ds(0, HALF)],
        send0, load_sems.at[0])
    load_b = pltpu.make_async_copy(
        partial_ref.at[pl.ds(my * CHUNK, CHUNK), pl.ds(HALF, HALF)],
        send1, load_sems.at[1])
    load_a.start()
    load_b.start()

    barrier = pltpu.get_barrier_semaphore()
    for nbr in (left, right):
        pl.semaphore_signal(barrier, inc=1, device_id=(nbr,),
                            device_id_type=pl.DeviceIdType.MESH)
    pl.semaphore_wait(barrier, 2)

    def rdmas():
        r0 = pltpu.make_async_remote_copy(
            src_ref=send0, dst_ref=work0,
            send_sem=send_sems.at[0], recv_sem=recv_sems.at[0],
            device_id=(right,), device_id_type=pl.DeviceIdType.MESH)
        r1 = pltpu.make_async_remote_copy(
            src_ref=send1, dst_ref=work1,
            send_sem=send_sems.at[1], recv_sem=recv_sems.at[1],
            device_id=(left,), device_id_type=pl.DeviceIdType.MESH)
        return r0, r1

    def wait_credits():
        pl.semaphore_wait(credit0, 1)
        pl.semaphore_wait(credit1, 1)

    def grant_credits():
        pl.semaphore_signal(credit0, inc=1, device_id=(left,),
                            device_id_type=pl.DeviceIdType.MESH)
        pl.semaphore_signal(credit1, inc=1, device_id=(right,),
                            device_id_type=pl.DeviceIdType.MESH)

    pending = {0: [], 1: []}

    def store_half(src_ref, c, d):
        outst = (outst0, outst1)[d]
        for sub in range(CHUNK // SUB):
            if pending[d]:
                pending[d].pop().wait()
            outst[...] = src_ref[pl.ds(sub * SUB, SUB), :].astype(jnp.float32)
            st = pltpu.make_async_copy(
                outst,
                out_ref.at[pl.ds(c * CHUNK + sub * SUB, SUB),
                           pl.ds(d * HALF, HALF)],
                store_sems.at[d])
            st.start()
            pending[d].append(st)

    load_a.wait()
    load_b.wait()

    for s in range(N_DEV - 1):
        c0 = (my - s - 1) % N_DEV
        c1 = (my + s + 1) % N_DEV
        if s >= 1:
            wait_credits()
        r0, r1 = rdmas()
        r0.start()
        r1.start()
        load_a = pltpu.make_async_copy(
            partial_ref.at[pl.ds(c0 * CHUNK, CHUNK), pl.ds(0, HALF)],
            mine0, load_sems.at[0])
        load_b = pltpu.make_async_copy(
            partial_ref.at[pl.ds(c1 * CHUNK, CHUNK), pl.ds(HALF, HALF)],
            mine1, load_sems.at[1])
        load_a.start()
        load_b.start()
        load_a.wait()
        load_b.wait()
        r0.wait()
        r1.wait()
        send0[...] = (work0[...].astype(jnp.float32)
                      + mine0[...].astype(jnp.float32)).astype(jnp.bfloat16)
        send1[...] = (work1[...].astype(jnp.float32)
                      + mine1[...].astype(jnp.float32)).astype(jnp.bfloat16)
        grant_credits()

    own0 = (my + 1) % N_DEV
    own1 = (my - 1) % N_DEV
    y0 = send0[...].astype(jnp.float32)
    send0[...] = (y0 * jax.nn.sigmoid(y0)).astype(jnp.bfloat16)
    y1 = send1[...].astype(jnp.float32)
    send1[...] = (y1 * jax.nn.sigmoid(y1)).astype(jnp.bfloat16)

    for s in range(N_DEV - 1):
        wait_credits()
        r0, r1 = rdmas()
        r0.start()
        r1.start()
        if s == 0:
            store_half(send0, own0, 0)
            store_half(send1, own1, 1)
        else:
            store_half(send0, (my - s + 1) % N_DEV, 0)
            store_half(send1, (my + s - 1) % N_DEV, 1)
        r0.wait()
        r1.wait()
        send0[...] = work0[...]
        send1[...] = work1[...]
        if s < N_DEV - 2:
            grant_credits()
    store_half(send0, (my - 2) % N_DEV, 0)
    store_half(send1, (my + 2) % N_DEV, 1)
    pending[0].pop().wait()
    pending[1].pop().wait()


def kernel(x, w_mat):
    partial = jnp.dot(
        x, w_mat, preferred_element_type=jnp.float32
    ).astype(jnp.bfloat16)

    return pl.pallas_call(
        _body,
        out_shape=jax.ShapeDtypeStruct((M, N), jnp.float32),
        in_specs=[pl.BlockSpec(memory_space=pl.ANY)],
        out_specs=pl.BlockSpec(memory_space=pl.ANY),
        scratch_shapes=[
            pltpu.VMEM((CHUNK, HALF), jnp.bfloat16),
            pltpu.VMEM((CHUNK, HALF), jnp.bfloat16),
            pltpu.VMEM((CHUNK, HALF), jnp.bfloat16),
            pltpu.VMEM((CHUNK, HALF), jnp.bfloat16),
            pltpu.VMEM((CHUNK, HALF), jnp.bfloat16),
            pltpu.VMEM((CHUNK, HALF), jnp.bfloat16),
            pltpu.VMEM((SUB, HALF), jnp.float32),
            pltpu.VMEM((SUB, HALF), jnp.float32),
            pltpu.SemaphoreType.DMA((2,)),
            pltpu.SemaphoreType.DMA((2,)),
            pltpu.SemaphoreType.DMA((2,)),
            pltpu.SemaphoreType.DMA((2,)),
            pltpu.SemaphoreType.REGULAR,
            pltpu.SemaphoreType.REGULAR,
        ],
        compiler_params=pltpu.CompilerParams(
            collective_id=0,
            vmem_limit_bytes=64 * 1024 * 1024,
        ),
    )(partial)


# device time: 824041 ns/iter; 1.7310x vs baseline; 1.0562x over previous
import jax
import jax.numpy as jnp
from jax import lax
from jax.experimental import pallas as pl
from jax.experimental.pallas import tpu as pltpu

N_DEV = 4
M = 8192
N = 4096
HALF = N // 2
CHUNK = M // N_DEV


def _body(partial_ref, out_ref,
          work0, work1, send0, send1, mine0, mine1,
          send_sems, recv_sems, load_sems, store_sems, credit0, credit1):
    my = lax.axis_index("i")
    left = (my - 1) % N_DEV
    right = (my + 1) % N_DEV

    load_a = pltpu.make_async_copy(
        partial_ref.at[pl.ds(my * CHUNK, CHUNK), pl.ds(0, HALF)],
        send0, load_sems.at[0])
    load_b = pltpu.make_async_copy(
        partial_ref.at[pl.ds(my * CHUNK, CHUNK), pl.ds(HALF, HALF)],
        send1, load_sems.at[1])
    load_a.start()
    load_b.start()

    barrier = pltpu.get_barrier_semaphore()
    for nbr in (left, right):
        pl.semaphore_signal(barrier, inc=1, device_id=(nbr,),
                            device_id_type=pl.DeviceIdType.MESH)
    pl.semaphore_wait(barrier, 2)

    def rdmas():
        r0 = pltpu.make_async_remote_copy(
            src_ref=send0, dst_ref=work0,
            send_sem=send_sems.at[0], recv_sem=recv_sems.at[0],
            device_id=(right,), device_id_type=pl.DeviceIdType.MESH)
        r1 = pltpu.make_async_remote_copy(
            src_ref=send1, dst_ref=work1,
            send_sem=send_sems.at[1], recv_sem=recv_sems.at[1],
            device_id=(left,), device_id_type=pl.DeviceIdType.MESH)
        return r0, r1

    def wait_credits():
        pl.semaphore_wait(credit0, 1)
        pl.semaphore_wait(credit1, 1)

    def grant_credits():
        pl.semaphore_signal(credit0, inc=1, device_id=(left,),
                            device_id_type=pl.DeviceIdType.MESH)
        pl.semaphore_signal(credit1, inc=1, device_id=(right,),
                            device_id_type=pl.DeviceIdType.MESH)

    pending = {0: [], 1: []}

    def store_half(src_ref, c, d):
        if pending[d]:
            pending[d].pop().wait()
        st = pltpu.make_async_copy(
            src_ref,
            out_ref.at[pl.ds(c * CHUNK, CHUNK), pl.ds(d * HALF, HALF)],
            store_sems.at[d])
        st.start()
        pending[d].append(st)

    load_a.wait()
    load_b.wait()

    for s in range(N_DEV - 1):
        c0 = (my - s - 1) % N_DEV
        c1 = (my + s + 1) % N_DEV
        if s >= 1:
            wait_credits()
        r0, r1 = rdmas()
        r0.start()
        r1.start()
        load_a = pltpu.make_async_copy(
            partial_ref.at[pl.ds(c0 * CHUNK, CHUNK), pl.ds(0, HALF)],
            mine0, load_sems.at[0])
        load_b = pltpu.make_async_copy(
            partial_ref.at[pl.ds(c1 * CHUNK, CHUNK), pl.ds(HALF, HALF)],
            mine1, load_sems.at[1])
        load_a.start()
        load_b.start()
        load_a.wait()
        load_b.wait()
        r0.wait()
        r1.wait()
        send0[...] = (work0[...].astype(jnp.float32)
                      + mine0[...].astype(jnp.float32)).astype(jnp.bfloat16)
        send1[...] = (work1[...].astype(jnp.float32)
                      + mine1[...].astype(jnp.float32)).astype(jnp.bfloat16)
        grant_credits()

    own0 = (my + 1) % N_DEV
    own1 = (my - 1) % N_DEV
    y0 = send0[...].astype(jnp.float32)
    send0[...] = (y0 * jax.nn.sigmoid(y0)).astype(jnp.bfloat16)
    y1 = send1[...].astype(jnp.float32)
    send1[...] = (y1 * jax.nn.sigmoid(y1)).astype(jnp.bfloat16)

    for s in range(N_DEV - 1):
        wait_credits()
        r0, r1 = rdmas()
        r0.start()
        r1.start()
        if s == 0:
            store_half(send0, own0, 0)
            store_half(send1, own1, 1)
        else:
            store_half(send0, (my - s + 1) % N_DEV, 0)
            store_half(send1, (my + s - 1) % N_DEV, 1)
        r0.wait()
        r1.wait()
        if pending[0]:
            pending[0].pop().wait()
        if pending[1]:
            pending[1].pop().wait()
        send0[...] = work0[...]
        send1[...] = work1[...]
        if s < N_DEV - 2:
            grant_credits()
    store_half(send0, (my - 2) % N_DEV, 0)
    store_half(send1, (my + 2) % N_DEV, 1)
    pending[0].pop().wait()
    pending[1].pop().wait()


def kernel(x, w_mat):
    partial = jnp.dot(
        x.astype(jnp.bfloat16), w_mat.astype(jnp.bfloat16),
        preferred_element_type=jnp.float32,
    ).astype(jnp.bfloat16)

    return pl.pallas_call(
        _body,
        out_shape=jax.ShapeDtypeStruct((M, N), jnp.bfloat16),
        in_specs=[pl.BlockSpec(memory_space=pl.ANY)],
        out_specs=pl.BlockSpec(memory_space=pl.ANY),
        scratch_shapes=[
            pltpu.VMEM((CHUNK, HALF), jnp.bfloat16),
            pltpu.VMEM((CHUNK, HALF), jnp.bfloat16),
            pltpu.VMEM((CHUNK, HALF), jnp.bfloat16),
            pltpu.VMEM((CHUNK, HALF), jnp.bfloat16),
            pltpu.VMEM((CHUNK, HALF), jnp.bfloat16),
            pltpu.VMEM((CHUNK, HALF), jnp.bfloat16),
            pltpu.SemaphoreType.DMA((2,)),
            pltpu.SemaphoreType.DMA((2,)),
            pltpu.SemaphoreType.DMA((2,)),
            pltpu.SemaphoreType.DMA((2,)),
            pltpu.SemaphoreType.REGULAR,
            pltpu.SemaphoreType.REGULAR,
        ],
        compiler_params=pltpu.CompilerParams(
            collective_id=0,
            vmem_limit_bytes=64 * 1024 * 1024,
        ),
    )(partial)


# device time: 717425 ns/iter; 1.9883x vs baseline; 1.1486x over previous
import jax
import jax.numpy as jnp
from jax import lax
from jax.experimental import pallas as pl
from jax.experimental.pallas import tpu as pltpu

jax.config.update("jax_compilation_cache_dir", "/tmp/jaxcache_scband")
jax.config.update("jax_persistent_cache_min_compile_time_secs", 1.0)

N_DEV = 4
M = 8192
K = 2048
N = 4096
HALF = N // 2
CHUNK = M // N_DEV
RCH = CHUNK // 2
SUBR = 512
N_ROUND = 2


def _body(x_ref, w_ref, out_ref,
          buf_a0, buf_b0, buf_a1, buf_b1, work0, work1,
          xbuf0, xbuf1, wh0, wh1,
          send_sems, recv_sems, load_sems, store_sems, credit0, credit1):
    my = lax.axis_index("i")
    left = (my - 1) % N_DEV
    right = (my + 1) % N_DEV

    work = (work0, work1)
    xbuf = (xbuf0, xbuf1)
    wh = (wh0, wh1)
    credit = (credit0, credit1)
    bufs = [[buf_a0, buf_b0], [buf_a1, buf_b1]]
    send_to = (right, left)
    recv_frm = (left, right)

    def start_send(d):
        r = pltpu.make_async_remote_copy(
            src_ref=bufs[d][0], dst_ref=work[d],
            send_sem=send_sems.at[d], recv_sem=recv_sems.at[d],
            device_id=(send_to[d],), device_id_type=pl.DeviceIdType.MESH)
        r.start()
        return r

    def grant(d):
        pl.semaphore_signal(credit[d], inc=1, device_id=(recv_frm[d],),
                            device_id_type=pl.DeviceIdType.MESH)

    def load_x(rows_base, d):
        ld = pltpu.make_async_copy(
            x_ref.at[pl.ds(rows_base, RCH), :], xbuf[d], load_sems.at[d])
        ld.start()
        return ld

    def dots_into(tgt_ref, d, src_x):
        for sub in range(RCH // SUBR):
            tgt_ref[pl.ds(sub * SUBR, SUBR), :] = jnp.dot(
                src_x[pl.ds(sub * SUBR, SUBR), :], wh[d][...],
                preferred_element_type=jnp.float32,
            ).astype(jnp.bfloat16)

    pending = {0: [], 1: []}

    def store_from(src_ref, c, r, d):
        if pending[d]:
            pending[d].pop().wait()
        st = pltpu.make_async_copy(
            src_ref,
            out_ref.at[pl.ds(c * CHUNK + r * RCH, RCH),
                       pl.ds(d * HALF, HALF)],
            store_sems.at[d])
        st.start()
        pending[d].append(st)

    wl0 = pltpu.make_async_copy(
        w_ref.at[:, pl.ds(0, HALF)], wh0, load_sems.at[0])
    wl1 = pltpu.make_async_copy(
        w_ref.at[:, pl.ds(HALF, HALF)], wh1, load_sems.at[1])
    wl0.start()
    wl1.start()

    barrier = pltpu.get_barrier_semaphore()
    for nbr in (left, right):
        pl.semaphore_signal(barrier, inc=1, device_id=(nbr,),
                            device_id_type=pl.DeviceIdType.MESH)

    wl0.wait()
    wl1.wait()
    lx = load_x(my * CHUNK, 0)
    lx.wait()
    dots_into(bufs[0][0], 0, xbuf[0])
    dots_into(bufs[1][0], 1, xbuf[0])

    pl.semaphore_wait(barrier, 2)

    first_send = [True]
    for rnd in range(N_ROUND):
        for s in range(N_DEV - 1):
            if not first_send[0]:
                pl.semaphore_wait(credit0, 1)
                pl.semaphore_wait(credit1, 1)
            first_send[0] = False
            rd = [start_send(0), start_send(1)]
            c_recv = ((my - s - 1) % N_DEV, (my + s + 1) % N_DEV)
            lds = [load_x(c_recv[0] * CHUNK + rnd * RCH, 0),
                   load_x(c_recv[1] * CHUNK + rnd * RCH, 1)]
            for d in (0, 1):
                lds[d].wait()
                dots_into(bufs[d][1], d, xbuf[d])
            for d in (0, 1):
                rd[d].wait()
                bufs[d][1][...] = (
                    bufs[d][1][...].astype(jnp.float32)
                    + work[d][...].astype(jnp.float32)).astype(jnp.bfloat16)
                grant(d)
                bufs[d].reverse()

        own = ((my + 1) % N_DEV, (my - 1) % N_DEV)
        for d in (0, 1):
            y = bufs[d][0][...].astype(jnp.float32)
            bufs[d][0][...] = (y * jax.nn.sigmoid(y)).astype(jnp.bfloat16)

        for s in range(N_DEV - 1):
            pl.semaphore_wait(credit0, 1)
            pl.semaphore_wait(credit1, 1)
            rd = [start_send(0), start_send(1)]
            for d in (0, 1):
                sgn = 1 if d == 0 else -1
                c_out = own[d] if s == 0 else (my - sgn * (s - 1)) % N_DEV
                store_from(bufs[d][0], c_out, rnd, d)
            if rnd == 0 and s == N_DEV - 2:
                lx = load_x(my * CHUNK + RCH, 0)
                lx.wait()
                for d in (0, 1):
                    if pending[d]:
                        pending[d].pop().wait()
                    dots_into(bufs[d][1], d, xbuf[0])
            for d in (0, 1):
                rd[d].wait()
                sgn = 1 if d == 0 else -1
                if s < N_DEV - 2:
                    bufs[d][1][...] = work[d][...]
                    grant(d)
                    bufs[d].reverse()
                else:
                    c_last = (my - sgn * s) % N_DEV
                    store_from(work[d], c_last, rnd, d)
                    if rnd == 0:
                        pending[d].pop().wait()
                        grant(d)
                        bufs[d].reverse()
    for d in (0, 1):
        while pending[d]:
            pending[d].pop().wait()


def kernel(x, w_mat):
    x = x.astype(jnp.bfloat16)
    w_mat = w_mat.astype(jnp.bfloat16)
    return pl.pallas_call(
        _body,
        out_shape=jax.ShapeDtypeStruct((M, N), jnp.bfloat16),
        in_specs=[pl.BlockSpec(memory_space=pl.ANY),
                  pl.BlockSpec(memory_space=pl.ANY)],
        out_specs=pl.BlockSpec(memory_space=pl.ANY),
        scratch_shapes=[
            pltpu.VMEM((RCH, HALF), jnp.bfloat16),
            pltpu.VMEM((RCH, HALF), jnp.bfloat16),
            pltpu.VMEM((RCH, HALF), jnp.bfloat16),
            pltpu.VMEM((RCH, HALF), jnp.bfloat16),
            pltpu.VMEM((RCH, HALF), jnp.bfloat16),
            pltpu.VMEM((RCH, HALF), jnp.bfloat16),
            pltpu.VMEM((RCH, K), jnp.bfloat16),
            pltpu.VMEM((RCH, K), jnp.bfloat16),
            pltpu.VMEM((K, HALF), jnp.bfloat16),
            pltpu.VMEM((K, HALF), jnp.bfloat16),
            pltpu.SemaphoreType.DMA((2,)),
            pltpu.SemaphoreType.DMA((2,)),
            pltpu.SemaphoreType.DMA((2,)),
            pltpu.SemaphoreType.DMA((2,)),
            pltpu.SemaphoreType.REGULAR,
            pltpu.SemaphoreType.REGULAR,
        ],
        compiler_params=pltpu.CompilerParams(
            collective_id=0,
            vmem_limit_bytes=64 * 1024 * 1024,
        ),
    )(x, w_mat)


# device time: 683939 ns/iter; 2.0856x vs baseline; 1.0490x over previous
import jax
import jax.numpy as jnp
from jax import lax
from jax.experimental import pallas as pl
from jax.experimental.pallas import tpu as pltpu

jax.config.update("jax_compilation_cache_dir", "/tmp/jaxcache_scband")
jax.config.update("jax_persistent_cache_min_compile_time_secs", 1.0)

N_DEV = 4
M = 8192
K = 2048
N = 4096
HALF = N // 2
CHUNK = M // N_DEV
RCH = CHUNK // 2
SUBR = 512
N_ROUND = 2


def _body(x_ref, w_ref, out_ref,
          buf_a0, buf_b0, buf_a1, buf_b1, work0, work1,
          xbuf0, xbuf1, wh0, wh1, f32st,
          send_sems, recv_sems, load_sems, store_sems, credit0, credit1):
    my = lax.axis_index("i")
    left = (my - 1) % N_DEV
    right = (my + 1) % N_DEV

    work = (work0, work1)
    xbuf = (xbuf0, xbuf1)
    wh = (wh0, wh1)
    credit = (credit0, credit1)
    bufs = [[buf_a0, buf_b0], [buf_a1, buf_b1]]
    send_to = (right, left)
    recv_frm = (left, right)

    def start_send(d):
        r = pltpu.make_async_remote_copy(
            src_ref=bufs[d][0], dst_ref=work[d],
            send_sem=send_sems.at[d], recv_sem=recv_sems.at[d],
            device_id=(send_to[d],), device_id_type=pl.DeviceIdType.MESH)
        r.start()
        return r

    def grant(d):
        pl.semaphore_signal(credit[d], inc=1, device_id=(recv_frm[d],),
                            device_id_type=pl.DeviceIdType.MESH)

    def load_x(rows_base, d):
        for strip in range(RCH // SUBR):
            ld = pltpu.make_async_copy(
                x_ref.at[pl.ds(rows_base + strip * SUBR, SUBR), :],
                f32st, load_sems.at[0])
            ld.start()
            ld.wait()
            xbuf[d][pl.ds(strip * SUBR, SUBR), :] = (
                f32st[...].astype(jnp.bfloat16))

    def dots_into(tgt_ref, d, src_x):
        for sub in range(RCH // SUBR):
            tgt_ref[pl.ds(sub * SUBR, SUBR), :] = jnp.dot(
                src_x[pl.ds(sub * SUBR, SUBR), :], wh[d][...],
                preferred_element_type=jnp.float32,
            ).astype(jnp.bfloat16)

    pending = {0: [], 1: []}

    def store_from(src_ref, c, r, d):
        if pending[d]:
            pending[d].pop().wait()
        st = pltpu.make_async_copy(
            src_ref,
            out_ref.at[pl.ds(c * CHUNK + r * RCH, RCH),
                       pl.ds(d * HALF, HALF)],
            store_sems.at[d])
        st.start()
        pending[d].append(st)

    barrier = pltpu.get_barrier_semaphore()
    for nbr in (left, right):
        pl.semaphore_signal(barrier, inc=1, device_id=(nbr,),
                            device_id_type=pl.DeviceIdType.MESH)

    for d in (0, 1):
        for strip in range(K // SUBR):
            ld = pltpu.make_async_copy(
                w_ref.at[pl.ds(strip * SUBR, SUBR), pl.ds(d * HALF, HALF)],
                f32st, load_sems.at[0])
            ld.start()
            ld.wait()
            wh[d][pl.ds(strip * SUBR, SUBR), :] = (
                f32st[...].astype(jnp.bfloat16))
    load_x(my * CHUNK, 0)
    dots_into(bufs[0][0], 0, xbuf[0])
    dots_into(bufs[1][0], 1, xbuf[0])

    pl.semaphore_wait(barrier, 2)

    first_send = [True]
    for rnd in range(N_ROUND):
        for s in range(N_DEV - 1):
            if not first_send[0]:
                pl.semaphore_wait(credit0, 1)
                pl.semaphore_wait(credit1, 1)
            first_send[0] = False
            rd = [start_send(0), start_send(1)]
            c_recv = ((my - s - 1) % N_DEV, (my + s + 1) % N_DEV)
            for d in (0, 1):
                load_x(c_recv[d] * CHUNK + rnd * RCH, d)
                dots_into(bufs[d][1], d, xbuf[d])
            for d in (0, 1):
                rd[d].wait()
                bufs[d][1][...] = (
                    bufs[d][1][...].astype(jnp.float32)
                    + work[d][...].astype(jnp.float32)).astype(jnp.bfloat16)
                grant(d)
                bufs[d].reverse()

        own = ((my + 1) % N_DEV, (my - 1) % N_DEV)
        for d in (0, 1):
            y = bufs[d][0][...].astype(jnp.float32)
            bufs[d][0][...] = (y * jax.nn.sigmoid(y)).astype(jnp.bfloat16)

        for s in range(N_DEV - 1):
            pl.semaphore_wait(credit0, 1)
            pl.semaphore_wait(credit1, 1)
            rd = [start_send(0), start_send(1)]
            for d in (0, 1):
                sgn = 1 if d == 0 else -1
                c_out = own[d] if s == 0 else (my - sgn * (s - 1)) % N_DEV
                store_from(bufs[d][0], c_out, rnd, d)
            if rnd == 0 and s == N_DEV - 2:
                load_x(my * CHUNK + RCH, 0)
                for d in (0, 1):
                    if pending[d]:
                        pending[d].pop().wait()
                    dots_into(bufs[d][1], d, xbuf[0])
            for d in (0, 1):
                rd[d].wait()
                sgn = 1 if d == 0 else -1
                if s < N_DEV - 2:
                    bufs[d][1][...] = work[d][...]
                    grant(d)
                    bufs[d].reverse()
                else:
                    c_last = (my - sgn * s) % N_DEV
                    store_from(work[d], c_last, rnd, d)
                    if rnd == 0:
                        pending[d].pop().wait()
                        grant(d)
                        bufs[d].reverse()
    for d in (0, 1):
        while pending[d]:
            pending[d].pop().wait()


def kernel(x, w_mat):
    return pl.pallas_call(
        _body,
        out_shape=jax.ShapeDtypeStruct((M, N), jnp.bfloat16),
        in_specs=[pl.BlockSpec(memory_space=pl.ANY),
                  pl.BlockSpec(memory_space=pl.ANY)],
        out_specs=pl.BlockSpec(memory_space=pl.ANY),
        scratch_shapes=[
            pltpu.VMEM((RCH, HALF), jnp.bfloat16),
            pltpu.VMEM((RCH, HALF), jnp.bfloat16),
            pltpu.VMEM((RCH, HALF), jnp.bfloat16),
            pltpu.VMEM((RCH, HALF), jnp.bfloat16),
            pltpu.VMEM((RCH, HALF), jnp.bfloat16),
            pltpu.VMEM((RCH, HALF), jnp.bfloat16),
            pltpu.VMEM((RCH, K), jnp.bfloat16),
            pltpu.VMEM((RCH, K), jnp.bfloat16),
            pltpu.VMEM((K, HALF), jnp.bfloat16),
            pltpu.VMEM((K, HALF), jnp.bfloat16),
            pltpu.VMEM((SUBR, K), jnp.float32),
            pltpu.SemaphoreType.DMA((2,)),
            pltpu.SemaphoreType.DMA((2,)),
            pltpu.SemaphoreType.DMA((2,)),
            pltpu.SemaphoreType.DMA((2,)),
            pltpu.SemaphoreType.REGULAR,
            pltpu.SemaphoreType.REGULAR,
        ],
        compiler_params=pltpu.CompilerParams(
            collective_id=0,
            vmem_limit_bytes=64 * 1024 * 1024,
        ),
    )(x, w_mat)
